# baseline (speedup 1.0000x reference)
import functools

import jax
import jax.numpy as jnp
from jax import lax
from jax.experimental import pallas as pl
from jax.experimental.pallas import tpu as pltpu
from jax.experimental.pallas import tpu_sc as plsc


_CHUNK = 32


def _sc_gather_pair(table_s, table_e, idx_s, idx_e):
    n = idx_s.shape[0]
    h = table_s.shape[1]
    info = plsc.get_sparse_core_info()
    nc, ns = info.num_cores, info.num_subcores
    nw = nc * ns
    per_w = n // nw
    nchunks = per_w // _CHUNK
    mesh = plsc.VectorSubcoreMesh(core_axis_name="core", subcore_axis_name="subcore")

    @functools.partial(
        pl.kernel,
        out_type=(
            jax.ShapeDtypeStruct((n, h), jnp.float32),
            jax.ShapeDtypeStruct((n, h), jnp.float32),
        ),
        mesh=mesh,
        scratch_types=[
            pltpu.VMEM((per_w,), jnp.int32),
            pltpu.VMEM((per_w,), jnp.int32),
            pltpu.VMEM((_CHUNK, h), jnp.float32),
            pltpu.VMEM((_CHUNK, h), jnp.float32),
            pltpu.SemaphoreType.DMA,
            pltpu.SemaphoreType.DMA,
            pltpu.SemaphoreType.DMA,
            pltpu.SemaphoreType.DMA,
        ],
    )
    def gather_kernel(ts_hbm, te_hbm, is_hbm, ie_hbm, os_hbm, oe_hbm,
                      idx_s_v, idx_e_v, rows0, rows1, gsem0, gsem1,
                      wsem0, wsem1):
        wid = lax.axis_index("subcore") * nc + lax.axis_index("core")
        base = wid * per_w
        i0 = pltpu.async_copy(is_hbm.at[pl.ds(base, per_w)], idx_s_v, gsem0)
        i1 = pltpu.async_copy(ie_hbm.at[pl.ds(base, per_w)], idx_e_v, gsem1)
        i0.wait()
        i1.wait()

        rows = (rows0, rows1)
        gsems = (gsem0, gsem1)
        wsems = (wsem0, wsem1)
        total = 2 * nchunks

        def chunk_src(k):
            if k < nchunks:
                return ts_hbm, idx_s_v, os_hbm, k * _CHUNK
            c = k - nchunks
            return te_hbm, idx_e_v, oe_hbm, c * _CHUNK

        gpend = [None, None]
        wpend = [None, None]
        for k in range(total + 1):
            if k < total:
                b = k % 2
                if wpend[b] is not None:
                    wpend[b].wait()
                t_hbm, i_v, _, off = chunk_src(k)
                gpend[b] = pltpu.async_copy(
                    t_hbm.at[i_v.at[pl.ds(off, _CHUNK)]], rows[b], gsems[b]
                )
            if k >= 1:
                b = (k - 1) % 2
                gpend[b].wait()
                _, _, o_hbm, off = chunk_src(k - 1)
                wpend[b] = pltpu.async_copy(
                    rows[b], o_hbm.at[pl.ds(base + off, _CHUNK)], wsems[b]
                )
        for p in wpend:
            if p is not None:
                p.wait()

    return gather_kernel(table_s, table_e, idx_s, idx_e)


_T = 256


def _fused_body(inp_ref, ps_ref, pe_ref, w1_ref, w2_ref, wp_ref, b1_ref,
                b2_ref, bp_ref, out_ref, w2p_s, bpr_s):
    h = w2_ref.shape[0]

    @pl.when(pl.program_id(0) == 0)
    def _():
        w2p_s[...] = jnp.dot(
            w2_ref[...], wp_ref[h:, :], preferred_element_type=jnp.float32
        )
        bpr_s[...] = (
            jnp.dot(b2_ref[...], wp_ref[h:, :], preferred_element_type=jnp.float32)
            + bp_ref[...]
        )

    acc = jnp.dot(ps_ref[...], w1_ref[:h, :], preferred_element_type=jnp.float32)
    acc += jnp.dot(pe_ref[...], w1_ref[h:, :], preferred_element_type=jnp.float32)
    acc += b1_ref[...]
    acc = jnp.where(acc >= 0, acc, 0.01 * acc)
    out = jnp.dot(acc, w2p_s[...], preferred_element_type=jnp.float32)
    out += jnp.dot(inp_ref[...], wp_ref[:h, :], preferred_element_type=jnp.float32)
    out_ref[...] = out + bpr_s[...]


def _fused_chain(inp2, ps, pe, w1, w2, wp, b1, b2, bp):
    n, h = inp2.shape
    blk = lambda i: (i, 0)
    fixed = lambda i: (0, 0)
    return pl.pallas_call(
        _fused_body,
        grid=(n // _T,),
        in_specs=[
            pl.BlockSpec((_T, h), blk),
            pl.BlockSpec((_T, h), blk),
            pl.BlockSpec((_T, h), blk),
            pl.BlockSpec((2 * h, h), fixed),
            pl.BlockSpec((h, h), fixed),
            pl.BlockSpec((2 * h, h), fixed),
            pl.BlockSpec((1, h), fixed),
            pl.BlockSpec((1, h), fixed),
            pl.BlockSpec((1, h), fixed),
        ],
        out_specs=pl.BlockSpec((_T, h), blk),
        out_shape=jax.ShapeDtypeStruct((n, h), jnp.float32),
        scratch_shapes=[
            pltpu.VMEM((h, h), jnp.float32),
            pltpu.VMEM((1, h), jnp.float32),
        ],
        compiler_params=pltpu.CompilerParams(
            dimension_semantics=("arbitrary",),
        ),
    )(inp2, ps, pe, w1, w2, wp, b1, b2, bp)


def kernel(inp, pos_s, pos_e, pe_s, pe_e, W1, b1, W2, b2, Wp, bp):
    B, L, H = inp.shape
    n = B * L
    inp2 = inp.reshape(n, H)
    ps, pe_g = _sc_gather_pair(pe_s, pe_e, pos_s.reshape(n), pos_e.reshape(n))
    out = _fused_chain(
        inp2, ps, pe_g, W1, W2, Wp,
        b1.reshape(1, H), b2.reshape(1, H), bp.reshape(1, H)
    )
    return out.reshape(B, L, H)


# speedup vs baseline: 1.0344x; 1.0344x over previous
import functools

import jax
import jax.numpy as jnp
from jax import lax
from jax.experimental import pallas as pl
from jax.experimental.pallas import tpu as pltpu
from jax.experimental.pallas import tpu_sc as plsc


_CHUNK = 32


def _sc_gather_pair(table_s, table_e, idx_s, idx_e):
    n = idx_s.shape[0]
    h = table_s.shape[1]
    info = plsc.get_sparse_core_info()
    nc, ns = info.num_cores, info.num_subcores
    nw = nc * ns
    per_w = n // nw
    nchunks = per_w // _CHUNK
    mesh = plsc.VectorSubcoreMesh(core_axis_name="core", subcore_axis_name="subcore")

    @functools.partial(
        pl.kernel,
        out_type=(
            jax.ShapeDtypeStruct((n, h), jnp.float32),
            jax.ShapeDtypeStruct((n, h), jnp.float32),
        ),
        mesh=mesh,
        scratch_types=[
            pltpu.VMEM((per_w,), jnp.int32),
            pltpu.VMEM((per_w,), jnp.int32),
            pltpu.VMEM((_CHUNK, h), jnp.float32),
            pltpu.VMEM((_CHUNK, h), jnp.float32),
            pltpu.VMEM((_CHUNK, h), jnp.float32),
            pltpu.SemaphoreType.DMA,
            pltpu.SemaphoreType.DMA,
            pltpu.SemaphoreType.DMA,
            pltpu.SemaphoreType.DMA,
            pltpu.SemaphoreType.DMA,
            pltpu.SemaphoreType.DMA,
        ],
    )
    def gather_kernel(ts_hbm, te_hbm, is_hbm, ie_hbm, os_hbm, oe_hbm,
                      idx_s_v, idx_e_v, rows0, rows1, rows2,
                      gsem0, gsem1, gsem2, wsem0, wsem1, wsem2):
        wid = lax.axis_index("subcore") * nc + lax.axis_index("core")
        base = wid * per_w
        i0 = pltpu.async_copy(is_hbm.at[pl.ds(base, per_w)], idx_s_v, gsem0)
        i1 = pltpu.async_copy(ie_hbm.at[pl.ds(base, per_w)], idx_e_v, gsem1)
        i0.wait()
        i1.wait()

        rows = (rows0, rows1, rows2)
        gsems = (gsem0, gsem1, gsem2)
        wsems = (wsem0, wsem1, wsem2)
        nbuf = 3
        total = 2 * nchunks

        def chunk_src(k):
            if k < nchunks:
                return ts_hbm, idx_s_v, os_hbm, k * _CHUNK
            c = k - nchunks
            return te_hbm, idx_e_v, oe_hbm, c * _CHUNK

        gpend = [None] * nbuf
        wpend = [None] * nbuf
        for k in range(total + 2):
            if k < total:
                b = k % nbuf
                if wpend[b] is not None:
                    wpend[b].wait()
                t_hbm, i_v, _, off = chunk_src(k)
                gpend[b] = pltpu.async_copy(
                    t_hbm.at[i_v.at[pl.ds(off, _CHUNK)]], rows[b], gsems[b]
                )
            if k >= 2:
                b = (k - 2) % nbuf
                gpend[b].wait()
                _, _, o_hbm, off = chunk_src(k - 2)
                wpend[b] = pltpu.async_copy(
                    rows[b], o_hbm.at[pl.ds(base + off, _CHUNK)], wsems[b]
                )
        for p in wpend:
            if p is not None:
                p.wait()

    return gather_kernel(table_s, table_e, idx_s, idx_e)


_T = 512


def _fused_body(inp_ref, ps_ref, pe_ref, w1_ref, w2_ref, wp_ref, b1_ref,
                b2_ref, bp_ref, out_ref, w2p_s, bpr_s):
    h = w2_ref.shape[0]

    @pl.when(pl.program_id(0) == 0)
    def _():
        w2p_s[...] = jnp.dot(
            w2_ref[...], wp_ref[h:, :], preferred_element_type=jnp.float32
        )
        bpr_s[...] = (
            jnp.dot(b2_ref[...], wp_ref[h:, :], preferred_element_type=jnp.float32)
            + bp_ref[...]
        )

    acc = jnp.dot(ps_ref[...], w1_ref[:h, :], preferred_element_type=jnp.float32)
    acc += jnp.dot(pe_ref[...], w1_ref[h:, :], preferred_element_type=jnp.float32)
    acc += b1_ref[...]
    acc = jnp.where(acc >= 0, acc, 0.01 * acc)
    out = jnp.dot(acc, w2p_s[...], preferred_element_type=jnp.float32)
    out += jnp.dot(inp_ref[...], wp_ref[:h, :], preferred_element_type=jnp.float32)
    out_ref[...] = out + bpr_s[...]


def _fused_chain(inp2, ps, pe, w1, w2, wp, b1, b2, bp):
    n, h = inp2.shape
    blk = lambda i: (i, 0)
    fixed = lambda i: (0, 0)
    return pl.pallas_call(
        _fused_body,
        grid=(n // _T,),
        in_specs=[
            pl.BlockSpec((_T, h), blk),
            pl.BlockSpec((_T, h), blk),
            pl.BlockSpec((_T, h), blk),
            pl.BlockSpec((2 * h, h), fixed),
            pl.BlockSpec((h, h), fixed),
            pl.BlockSpec((2 * h, h), fixed),
            pl.BlockSpec((1, h), fixed),
            pl.BlockSpec((1, h), fixed),
            pl.BlockSpec((1, h), fixed),
        ],
        out_specs=pl.BlockSpec((_T, h), blk),
        out_shape=jax.ShapeDtypeStruct((n, h), jnp.float32),
        scratch_shapes=[
            pltpu.VMEM((h, h), jnp.float32),
            pltpu.VMEM((1, h), jnp.float32),
        ],
        compiler_params=pltpu.CompilerParams(
            dimension_semantics=("arbitrary",),
        ),
    )(inp2, ps, pe, w1, w2, wp, b1, b2, bp)


def kernel(inp, pos_s, pos_e, pe_s, pe_e, W1, b1, W2, b2, Wp, bp):
    B, L, H = inp.shape
    n = B * L
    inp2 = inp.reshape(n, H)
    ps, pe_g = _sc_gather_pair(pe_s, pe_e, pos_s.reshape(n), pos_e.reshape(n))
    out = _fused_chain(
        inp2, ps, pe_g, W1, W2, Wp,
        b1.reshape(1, H), b2.reshape(1, H), bp.reshape(1, H)
    )
    return out.reshape(B, L, H)
